# Initial kernel scaffold; baseline (speedup 1.0000x reference)
import jax
import jax.numpy as jnp
from jax.experimental import pallas as pl


def kernel(cat0, cat1, cat2, Interaction, E_cat0, E_cat1, E_cat2, E_inter, W, b):
    raise NotImplementedError("write your pallas kernel here")


# speedup vs baseline: 4.2270x; 4.2270x over previous
import functools

import jax
import jax.numpy as jnp
from jax import lax
from jax.experimental import pallas as pl
from jax.experimental.pallas import tpu as pltpu
from jax.experimental.pallas import tpu_sc as plsc

NC = 2
NS = 16
NW = NC * NS
CHUNK = 128


def _sc_gather_body(e0, e1, e2, i0, i1, i2, g0, g1, g2, idx_v, rows_v, sem):
    n_per_w = i0.shape[0] // NW
    n_chunks = n_per_w // CHUNK
    wid = lax.axis_index("s") * NC + lax.axis_index("c")
    base = wid * n_per_w
    for e, i, g in ((e0, i0, g0), (e1, i1, g1), (e2, i2, g2)):
        pltpu.sync_copy(i.at[pl.ds(base, n_per_w)], idx_v)

        def body(j, carry, e=e, g=g):
            off = j * CHUNK
            pltpu.async_copy(e.at[idx_v.at[pl.ds(off, CHUNK)]], rows_v, sem).wait()
            pltpu.sync_copy(rows_v, g.at[pl.ds(base + off, CHUNK)])
            return carry

        lax.fori_loop(0, n_chunks, body, 0)


def _tc_proj_body(g0b, g1b, g2b, intb, w_ref, ei_ref, b_ref, out_ref):
    w = w_ref[...]
    d = g0b.shape[1]
    t3 = jnp.dot(ei_ref[...], w[3 * d:4 * d, :],
                 preferred_element_type=jnp.float32)
    acc = jnp.dot(g0b[...], w[:d, :], preferred_element_type=jnp.float32)
    acc += jnp.dot(g1b[...], w[d:2 * d, :], preferred_element_type=jnp.float32)
    acc += jnp.dot(g2b[...], w[2 * d:3 * d, :], preferred_element_type=jnp.float32)
    it = intb[0, 0, :].reshape(intb.shape[2], 1)
    acc += jnp.where(it == 0, t3[0:1, :],
                     jnp.where(it == 1, t3[1:2, :], t3[2:3, :]))
    out_ref[...] = acc + b_ref[...]


def kernel(cat0, cat1, cat2, Interaction, E_cat0, E_cat1, E_cat2, E_inter, W, b):
    B, L = cat0.shape
    N = B * L
    D = E_cat0.shape[1]
    HD = W.shape[1]
    n_per_w = N // NW

    idx0 = cat0.reshape(N).astype(jnp.int32)
    idx1 = cat1.reshape(N).astype(jnp.int32)
    idx2 = cat2.reshape(N).astype(jnp.int32)

    mesh = plsc.VectorSubcoreMesh(core_axis_name="c", subcore_axis_name="s")
    gath = pl.kernel(
        _sc_gather_body,
        out_type=[jax.ShapeDtypeStruct((N, D), jnp.float32)] * 3,
        mesh=mesh,
        scratch_types=[
            pltpu.VMEM((n_per_w,), jnp.int32),
            pltpu.VMEM((CHUNK, D), jnp.float32),
            pltpu.SemaphoreType.DMA,
        ],
    )
    g0, g1, g2 = gath(E_cat0, E_cat1, E_cat2, idx0, idx1, idx2)

    TB = 512
    nblk = N // TB
    inter3 = Interaction.reshape(nblk, 1, TB).astype(jnp.int32)
    ei_pad = jnp.zeros((8, D), jnp.float32).at[:3].set(E_inter)
    b2 = b.reshape(1, HD)

    X = pl.pallas_call(
        _tc_proj_body,
        grid=(nblk,),
        in_specs=[
            pl.BlockSpec((TB, D), lambda i: (i, 0)),
            pl.BlockSpec((TB, D), lambda i: (i, 0)),
            pl.BlockSpec((TB, D), lambda i: (i, 0)),
            pl.BlockSpec((1, 1, TB), lambda i: (i, 0, 0)),
            pl.BlockSpec((4 * D, HD), lambda i: (0, 0)),
            pl.BlockSpec((8, D), lambda i: (0, 0)),
            pl.BlockSpec((1, HD), lambda i: (0, 0)),
        ],
        out_specs=pl.BlockSpec((TB, HD), lambda i: (i, 0)),
        out_shape=jax.ShapeDtypeStruct((N, HD), jnp.float32),
        compiler_params=pltpu.CompilerParams(
            dimension_semantics=("arbitrary",)),
    )(g0, g1, g2, inter3, W, ei_pad, b2)

    return X.reshape(B, L, HD), B
